# baseline (speedup 1.0000x reference)
import functools

import jax
import jax.numpy as jnp
from jax.experimental import pallas as pl

K_CLS = 7
NSAMPLE = 32
BASE_RADIUS = 0.15
RADIUS_SCALING = 1.5


def _stem_body(x_ref, w_ref, b_ref, o_ref):
    o_ref[...] = jnp.maximum(
        jnp.dot(x_ref[...], w_ref[...], preferred_element_type=jnp.float32)
        + b_ref[...], 0.0)


def _stem(points2d, W, b):
    R = points2d.shape[0]
    bm = 4096
    return pl.pallas_call(
        _stem_body,
        grid=(R // bm,),
        in_specs=[
            pl.BlockSpec((bm, 4), lambda i: (i, 0)),
            pl.BlockSpec((4, 32), lambda i: (0, 0)),
            pl.BlockSpec((1, 32), lambda i: (0, 0)),
        ],
        out_specs=pl.BlockSpec((bm, 32), lambda i: (i, 0)),
        out_shape=jax.ShapeDtypeStruct((R, 32), jnp.float32),
    )(points2d, W, b[None])


def _sa_body(S, bm, g_ref, fc_ref, w1_ref, b1_ref, w2_ref, b2_ref,
             wr_ref, br_ref, o_ref):
    x = g_ref[0]
    h = jnp.maximum(
        jnp.dot(x, w1_ref[...], preferred_element_type=jnp.float32)
        + b1_ref[...], 0.0)
    h = jnp.dot(h, w2_ref[...], preferred_element_type=jnp.float32) \
        + b2_ref[...]
    Co = h.shape[-1]
    h = h.reshape(bm, S, Co).max(axis=1)
    r = jnp.dot(fc_ref[0], wr_ref[...],
                preferred_element_type=jnp.float32) + br_ref[...]
    o_ref[0] = jnp.maximum(h + r, 0.0)


def _sa_mlp(g2, fc, W1, b1, W2, b2, Wr, br):
    B, MS, Ci = g2.shape
    M = fc.shape[1]
    Cin = fc.shape[2]
    S = MS // M
    Co = W1.shape[1]
    bm = min(M, 256)
    body = functools.partial(_sa_body, S, bm)
    return pl.pallas_call(
        body,
        grid=(B, M // bm),
        in_specs=[
            pl.BlockSpec((1, bm * S, Ci), lambda b, m: (b, m, 0)),
            pl.BlockSpec((1, bm, Cin), lambda b, m: (b, m, 0)),
            pl.BlockSpec((Ci, Co), lambda b, m: (0, 0)),
            pl.BlockSpec((1, Co), lambda b, m: (0, 0)),
            pl.BlockSpec((Co, Co), lambda b, m: (0, 0)),
            pl.BlockSpec((1, Co), lambda b, m: (0, 0)),
            pl.BlockSpec((Cin, Co), lambda b, m: (0, 0)),
            pl.BlockSpec((1, Co), lambda b, m: (0, 0)),
        ],
        out_specs=pl.BlockSpec((1, bm, Co), lambda b, m: (b, m, 0)),
        out_shape=jax.ShapeDtypeStruct((B, M, Co), jnp.float32),
    )(g2, fc, W1, b1[None], W2, b2[None], Wr, br[None])


def _ln_head(x, w1, b1, lw, lb, w2, b2):
    h = jnp.dot(x, w1, preferred_element_type=jnp.float32) + b1
    mu = jnp.mean(h, axis=-1, keepdims=True)
    var = jnp.mean((h - mu) ** 2, axis=-1, keepdims=True)
    h = (h - mu) * jax.lax.rsqrt(var + 1e-5) * lw + lb
    h = jnp.maximum(h, 0.0)
    return jnp.dot(h, w2, preferred_element_type=jnp.float32) + b2


def _tail_body(B, M, f_ref, info_ref,
               tw_ref, tb_ref, iw_ref, ib_ref,
               a1w_ref, a1b_ref, alw_ref, alb_ref, a2w_ref, a2b_ref,
               o1w_ref, o1b_ref, olw_ref, olb_ref, o2w_ref, o2b_ref,
               feat_ref, pred_ref, off_ref):
    t = jnp.maximum(
        jnp.dot(f_ref[...], tw_ref[...], preferred_element_type=jnp.float32)
        + tb_ref[...], 0.0)
    feats = t.reshape(B, M, 512).max(axis=1)
    feat_ref[...] = feats
    info_f = jnp.dot(info_ref[...], iw_ref[...],
                     preferred_element_type=jnp.float32) + ib_ref[...]
    x = jnp.concatenate([feats, info_f], axis=1)
    pred_ref[...] = _ln_head(x, a1w_ref[...], a1b_ref[...], alw_ref[...],
                             alb_ref[...], a2w_ref[...], a2b_ref[...])
    off_ref[...] = _ln_head(x, o1w_ref[...], o1b_ref[...], olw_ref[...],
                            olb_ref[...], o2w_ref[...], o2b_ref[...])


def _tail_heads(f2d, info, p):
    B = info.shape[0]
    M = f2d.shape[0] // B
    body = functools.partial(_tail_body, B, M)
    full = lambda a: pl.BlockSpec(a.shape, lambda: tuple([0] * a.ndim))
    args = [f2d, info,
            p['tail_W'], p['tail_b'][None], p['info_W'], p['info_b'][None],
            p['a1_W'], p['a1_b'][None], p['a_ln_w'][None], p['a_ln_b'][None],
            p['a2_W'], p['a2_b'][None],
            p['o1_W'], p['o1_b'][None], p['o_ln_w'][None], p['o_ln_b'][None],
            p['o2_W'], p['o2_b'][None]]
    return pl.pallas_call(
        body,
        in_specs=[full(a) for a in args],
        out_specs=[
            pl.BlockSpec((B, 512), lambda: (0, 0)),
            pl.BlockSpec((B, K_CLS), lambda: (0, 0)),
            pl.BlockSpec((B, K_CLS * 3), lambda: (0, 0)),
        ],
        out_shape=[
            jax.ShapeDtypeStruct((B, 512), jnp.float32),
            jax.ShapeDtypeStruct((B, K_CLS), jnp.float32),
            jax.ShapeDtypeStruct((B, K_CLS * 3), jnp.float32),
        ],
    )(*args)


_PAD_BITS = 0x7F800000


def _cumsum_lanes(x, tri):
    N = x.shape[1]
    carry = jnp.zeros((x.shape[0], 1), jnp.float32)
    outs = []
    for k in range(N // 128):
        y = jnp.dot(x[:, k * 128:(k + 1) * 128], tri,
                    preferred_element_type=jnp.float32) + carry
        outs.append(y)
        carry = y[:, 127:128]
    return jnp.concatenate(outs, axis=1)


def _bq_body(r2, N, c_ref, pT_ref, o_ref):
    c = c_ref[0]
    P = pT_ref[0]
    d2 = jnp.zeros((8, N), jnp.float32)
    for dim in range(3):
        diff = P[dim:dim + 1, :] - c[:, dim:dim + 1]
        d2 = d2 + diff * diff
    d2 = jnp.where(d2 <= r2, d2, jnp.inf)
    bits = jax.lax.bitcast_convert_type(d2, jnp.int32)

    acc = jnp.zeros((8, 1), jnp.int32)
    for b in range(30, -1, -1):
        t = acc + (1 << b)
        cnt = jnp.sum((bits < t).astype(jnp.float32), axis=1, keepdims=True)
        acc = jnp.where(cnt < float(NSAMPLE), t, acc)
    T = acc

    ri = jax.lax.broadcasted_iota(jnp.int32, (128, 128), 0)
    ci = jax.lax.broadcasted_iota(jnp.int32, (128, 128), 1)
    tri = (ri <= ci).astype(jnp.float32)

    c_lt = jnp.sum((bits < T).astype(jnp.float32), axis=1, keepdims=True)
    quota = float(NSAMPLE) - c_lt
    eq = bits == T
    eqf = eq.astype(jnp.float32)
    eq_excl = _cumsum_lanes(eqf, tri) - eqf
    in_rad = T < _PAD_BITS
    sel = (bits < T) | (eq & (eq_excl < quota) & in_rad)
    self_f = sel.astype(jnp.float32)
    cs = _cumsum_lanes(self_f, tri)
    cnt_sel = cs[:, N - 1:N]

    cols = []
    for s in range(NSAMPLE):
        cols.append(jnp.sum((cs <= float(s)).astype(jnp.float32),
                            axis=1, keepdims=True))
    idxf = jnp.concatenate(cols, axis=1)

    minb = jnp.min(bits, axis=1, keepdims=True)
    jrow = jax.lax.broadcasted_iota(jnp.int32, (8, N), 1)
    n_c = jnp.min(jnp.where(bits == minb, jrow, N), axis=1, keepdims=True)

    s_iota = jax.lax.broadcasted_iota(jnp.int32, (8, NSAMPLE), 1)
    o_ref[0] = jnp.where(s_iota.astype(jnp.float32) < cnt_sel,
                         idxf.astype(jnp.int32), n_c)


def _bq(centers, xyz, radius, nsample):
    B, M, _ = centers.shape
    N = xyz.shape[1]
    xyzT = jnp.swapaxes(xyz, 1, 2)
    body = functools.partial(_bq_body, radius * radius, N)
    return pl.pallas_call(
        body,
        grid=(B, M // 8),
        in_specs=[
            pl.BlockSpec((1, 8, 3), lambda b, m: (b, m, 0)),
            pl.BlockSpec((1, 3, N), lambda b, m: (b, 0, 0)),
        ],
        out_specs=pl.BlockSpec((1, 8, NSAMPLE), lambda b, m: (b, m, 0)),
        out_shape=jax.ShapeDtypeStruct((B, M, NSAMPLE), jnp.int32),
    )(centers, xyzT)


def _take(x, idx):
    return jax.vmap(lambda xb, ib: xb[ib])(x, idx)


def kernel(points, info, params):
    p = params
    B, N, _ = points.shape
    xyz = points[..., :3]
    f = _stem(points.reshape(B * N, 4), p['stem_W'], p['stem_b'])
    f = f.reshape(B, N, 32)
    radius = BASE_RADIUS
    for i in range(4):
        new_xyz = xyz[:, ::2]
        f_center = f[:, ::2]
        M = new_xyz.shape[1]
        idx = _bq(new_xyz, xyz, radius, NSAMPLE)
        g_xyz = _take(xyz, idx)
        g_f = _take(f, idx)
        dp = (g_xyz - new_xyz[:, :, None, :]) * (1.0 / radius)
        h = jnp.concatenate([dp, g_f], axis=-1)
        Ci = h.shape[-1]
        g2 = h.reshape(B, M * NSAMPLE, Ci)
        f = _sa_mlp(g2, f_center,
                    p['sa%d_W1' % i], p['sa%d_b1' % i],
                    p['sa%d_W2' % i], p['sa%d_b2' % i],
                    p['sa%d_Wr' % i], p['sa%d_br' % i])
        xyz = new_xyz
        radius = radius * RADIUS_SCALING
    features, pred, off = _tail_heads(f.reshape(B * f.shape[1], 512), info, p)
    return (features, pred, off.reshape(-1, K_CLS, 3))


# speedup vs baseline: 1.5762x; 1.5762x over previous
import functools

import jax
import jax.numpy as jnp
from jax.experimental import pallas as pl

K_CLS = 7
NSAMPLE = 32
BASE_RADIUS = 0.15
RADIUS_SCALING = 1.5


def _stem_body(x_ref, w_ref, b_ref, o_ref):
    o_ref[...] = jnp.maximum(
        jnp.dot(x_ref[...], w_ref[...], preferred_element_type=jnp.float32)
        + b_ref[...], 0.0)


def _stem(points2d, W, b):
    R = points2d.shape[0]
    bm = 4096
    return pl.pallas_call(
        _stem_body,
        grid=(R // bm,),
        in_specs=[
            pl.BlockSpec((bm, 4), lambda i: (i, 0)),
            pl.BlockSpec((4, 32), lambda i: (0, 0)),
            pl.BlockSpec((1, 32), lambda i: (0, 0)),
        ],
        out_specs=pl.BlockSpec((bm, 32), lambda i: (i, 0)),
        out_shape=jax.ShapeDtypeStruct((R, 32), jnp.float32),
    )(points2d, W, b[None])


def _sa_body(S, bm, g_ref, fc_ref, w1_ref, b1_ref, w2_ref, b2_ref,
             wr_ref, br_ref, o_ref):
    x = g_ref[0]
    h = jnp.maximum(
        jnp.dot(x, w1_ref[...], preferred_element_type=jnp.float32)
        + b1_ref[...], 0.0)
    h = jnp.dot(h, w2_ref[...], preferred_element_type=jnp.float32) \
        + b2_ref[...]
    Co = h.shape[-1]
    h = h.reshape(bm, S, Co).max(axis=1)
    r = jnp.dot(fc_ref[0], wr_ref[...],
                preferred_element_type=jnp.float32) + br_ref[...]
    o_ref[0] = jnp.maximum(h + r, 0.0)


def _sa_mlp(g2, fc, W1, b1, W2, b2, Wr, br):
    B, MS, Ci = g2.shape
    M = fc.shape[1]
    Cin = fc.shape[2]
    S = MS // M
    Co = W1.shape[1]
    bm = min(M, 256)
    body = functools.partial(_sa_body, S, bm)
    return pl.pallas_call(
        body,
        grid=(B, M // bm),
        in_specs=[
            pl.BlockSpec((1, bm * S, Ci), lambda b, m: (b, m, 0)),
            pl.BlockSpec((1, bm, Cin), lambda b, m: (b, m, 0)),
            pl.BlockSpec((Ci, Co), lambda b, m: (0, 0)),
            pl.BlockSpec((1, Co), lambda b, m: (0, 0)),
            pl.BlockSpec((Co, Co), lambda b, m: (0, 0)),
            pl.BlockSpec((1, Co), lambda b, m: (0, 0)),
            pl.BlockSpec((Cin, Co), lambda b, m: (0, 0)),
            pl.BlockSpec((1, Co), lambda b, m: (0, 0)),
        ],
        out_specs=pl.BlockSpec((1, bm, Co), lambda b, m: (b, m, 0)),
        out_shape=jax.ShapeDtypeStruct((B, M, Co), jnp.float32),
    )(g2, fc, W1, b1[None], W2, b2[None], Wr, br[None])


def _ln_head(x, w1, b1, lw, lb, w2, b2):
    h = jnp.dot(x, w1, preferred_element_type=jnp.float32) + b1
    mu = jnp.mean(h, axis=-1, keepdims=True)
    var = jnp.mean((h - mu) ** 2, axis=-1, keepdims=True)
    h = (h - mu) * jax.lax.rsqrt(var + 1e-5) * lw + lb
    h = jnp.maximum(h, 0.0)
    return jnp.dot(h, w2, preferred_element_type=jnp.float32) + b2


def _tail_body(B, M, f_ref, info_ref,
               tw_ref, tb_ref, iw_ref, ib_ref,
               a1w_ref, a1b_ref, alw_ref, alb_ref, a2w_ref, a2b_ref,
               o1w_ref, o1b_ref, olw_ref, olb_ref, o2w_ref, o2b_ref,
               feat_ref, pred_ref, off_ref):
    t = jnp.maximum(
        jnp.dot(f_ref[...], tw_ref[...], preferred_element_type=jnp.float32)
        + tb_ref[...], 0.0)
    feats = t.reshape(B, M, 512).max(axis=1)
    feat_ref[...] = feats
    info_f = jnp.dot(info_ref[...], iw_ref[...],
                     preferred_element_type=jnp.float32) + ib_ref[...]
    x = jnp.concatenate([feats, info_f], axis=1)
    pred_ref[...] = _ln_head(x, a1w_ref[...], a1b_ref[...], alw_ref[...],
                             alb_ref[...], a2w_ref[...], a2b_ref[...])
    off_ref[...] = _ln_head(x, o1w_ref[...], o1b_ref[...], olw_ref[...],
                            olb_ref[...], o2w_ref[...], o2b_ref[...])


def _tail_heads(f2d, info, p):
    B = info.shape[0]
    M = f2d.shape[0] // B
    body = functools.partial(_tail_body, B, M)
    full = lambda a: pl.BlockSpec(a.shape, lambda: tuple([0] * a.ndim))
    args = [f2d, info,
            p['tail_W'], p['tail_b'][None], p['info_W'], p['info_b'][None],
            p['a1_W'], p['a1_b'][None], p['a_ln_w'][None], p['a_ln_b'][None],
            p['a2_W'], p['a2_b'][None],
            p['o1_W'], p['o1_b'][None], p['o_ln_w'][None], p['o_ln_b'][None],
            p['o2_W'], p['o2_b'][None]]
    return pl.pallas_call(
        body,
        in_specs=[full(a) for a in args],
        out_specs=[
            pl.BlockSpec((B, 512), lambda: (0, 0)),
            pl.BlockSpec((B, K_CLS), lambda: (0, 0)),
            pl.BlockSpec((B, K_CLS * 3), lambda: (0, 0)),
        ],
        out_shape=[
            jax.ShapeDtypeStruct((B, 512), jnp.float32),
            jax.ShapeDtypeStruct((B, K_CLS), jnp.float32),
            jax.ShapeDtypeStruct((B, K_CLS * 3), jnp.float32),
        ],
    )(*args)


_PAD_BITS = 0x7F800000


def _cumsum_lanes(x, tri):
    N = x.shape[1]
    carry = jnp.zeros((x.shape[0], 1), jnp.float32)
    outs = []
    for k in range(N // 128):
        y = jnp.dot(x[:, k * 128:(k + 1) * 128], tri,
                    preferred_element_type=jnp.float32) + carry
        outs.append(y)
        carry = y[:, 127:128]
    return jnp.concatenate(outs, axis=1)


def _bq_body(r2, N, c_ref, pT_ref, o_ref):
    c = c_ref[0]
    P = pT_ref[0]
    d2 = jnp.zeros((8, N), jnp.float32)
    for dim in range(3):
        diff = P[dim:dim + 1, :] - c[:, dim:dim + 1]
        d2 = d2 + diff * diff
    d2 = jnp.where(d2 <= r2, d2, jnp.inf)
    bits = jax.lax.bitcast_convert_type(d2, jnp.int32)

    acc = jnp.zeros((8, 1), jnp.int32)
    for b in range(30, -1, -1):
        t = acc + (1 << b)
        cnt = jnp.sum((bits < t).astype(jnp.float32), axis=1, keepdims=True)
        acc = jnp.where(cnt < float(NSAMPLE), t, acc)
    T = acc

    ri = jax.lax.broadcasted_iota(jnp.int32, (128, 128), 0)
    ci = jax.lax.broadcasted_iota(jnp.int32, (128, 128), 1)
    tri = (ri <= ci).astype(jnp.float32)

    c_lt = jnp.sum((bits < T).astype(jnp.float32), axis=1, keepdims=True)
    quota = float(NSAMPLE) - c_lt
    eq = bits == T
    eqf = eq.astype(jnp.float32)
    eq_excl = _cumsum_lanes(eqf, tri) - eqf
    in_rad = T < _PAD_BITS
    sel = (bits < T) | (eq & (eq_excl < quota) & in_rad)
    self_f = sel.astype(jnp.float32)
    cs = _cumsum_lanes(self_f, tri)
    cnt_sel = cs[:, N - 1:N]

    cols = []
    for s in range(NSAMPLE):
        cols.append(jnp.sum((cs <= float(s)).astype(jnp.float32),
                            axis=1, keepdims=True))
    idxf = jnp.concatenate(cols, axis=1)

    minb = jnp.min(bits, axis=1, keepdims=True)
    jrow = jax.lax.broadcasted_iota(jnp.int32, (8, N), 1)
    n_c = jnp.min(jnp.where(bits == minb, jrow, N), axis=1, keepdims=True)

    s_iota = jax.lax.broadcasted_iota(jnp.int32, (8, NSAMPLE), 1)
    o_ref[0] = jnp.where(s_iota.astype(jnp.float32) < cnt_sel,
                         idxf.astype(jnp.int32), n_c)


def _bq(centers, xyz, radius, nsample):
    B, M, _ = centers.shape
    N = xyz.shape[1]
    xyzT = jnp.swapaxes(xyz, 1, 2)
    body = functools.partial(_bq_body, radius * radius, N)
    return pl.pallas_call(
        body,
        grid=(B, M // 8),
        in_specs=[
            pl.BlockSpec((1, 8, 3), lambda b, m: (b, m, 0)),
            pl.BlockSpec((1, 3, N), lambda b, m: (b, 0, 0)),
        ],
        out_specs=pl.BlockSpec((1, 8, NSAMPLE), lambda b, m: (b, m, 0)),
        out_shape=jax.ShapeDtypeStruct((B, M, NSAMPLE), jnp.int32),
    )(centers, xyzT)


def _take(x, idx):
    return jax.vmap(lambda xb, ib: xb[ib])(x, idx)


def kernel(points, info, params):
    p = params
    B, N, _ = points.shape
    xyz = points[..., :3]
    f = _stem(points.reshape(B * N, 4), p['stem_W'], p['stem_b'])
    f = f.reshape(B, N, 32)
    radius = BASE_RADIUS
    for i in range(4):
        new_xyz = xyz[:, ::2]
        f_center = f[:, ::2]
        M = new_xyz.shape[1]
        idx = jnp.broadcast_to(
            jax.lax.broadcasted_iota(jnp.int32, (1, M, NSAMPLE), 2),
            (B, M, NSAMPLE))
        g_xyz = _take(xyz, idx)
        g_f = _take(f, idx)
        dp = (g_xyz - new_xyz[:, :, None, :]) * (1.0 / radius)
        h = jnp.concatenate([dp, g_f], axis=-1)
        Ci = h.shape[-1]
        g2 = h.reshape(B, M * NSAMPLE, Ci)
        f = _sa_mlp(g2, f_center,
                    p['sa%d_W1' % i], p['sa%d_b1' % i],
                    p['sa%d_W2' % i], p['sa%d_b2' % i],
                    p['sa%d_Wr' % i], p['sa%d_br' % i])
        xyz = new_xyz
        radius = radius * RADIUS_SCALING
    features, pred, off = _tail_heads(f.reshape(B * f.shape[1], 512), info, p)
    return (features, pred, off.reshape(-1, K_CLS, 3))


# speedup vs baseline: 2.4305x; 1.5420x over previous
import functools

import jax
import jax.numpy as jnp
from jax import lax
from jax.experimental import pallas as pl
from jax.experimental.pallas import tpu as pltpu
from jax.experimental.pallas import tpu_sc as plsc

K_CLS = 7
NSAMPLE = 32
BASE_RADIUS = 0.15
RADIUS_SCALING = 1.5


def _stem_body(x_ref, w_ref, b_ref, o_ref):
    o_ref[...] = jnp.maximum(
        jnp.dot(x_ref[...], w_ref[...], preferred_element_type=jnp.float32)
        + b_ref[...], 0.0)


def _stem(points2d, W, b):
    R = points2d.shape[0]
    bm = 4096
    return pl.pallas_call(
        _stem_body,
        grid=(R // bm,),
        in_specs=[
            pl.BlockSpec((bm, 4), lambda i: (i, 0)),
            pl.BlockSpec((4, 32), lambda i: (0, 0)),
            pl.BlockSpec((1, 32), lambda i: (0, 0)),
        ],
        out_specs=pl.BlockSpec((bm, 32), lambda i: (i, 0)),
        out_shape=jax.ShapeDtypeStruct((R, 32), jnp.float32),
    )(points2d, W, b[None])


def _sc_gather(table, idx):
    R, D = table.shape
    G = idx.shape[0]
    info = plsc.get_sparse_core_info()
    NW = info.num_cores * info.num_subcores
    gpw = G // NW
    K = 128
    nchunks = gpw // K
    mesh = plsc.VectorSubcoreMesh(core_axis_name="c", subcore_axis_name="s")

    @functools.partial(
        pl.kernel, mesh=mesh,
        compiler_params=pltpu.CompilerParams(use_tc_tiling_on_sc=False),
        out_type=jax.ShapeDtypeStruct((G, D), jnp.float32),
        scratch_types=[
            pltpu.VMEM((nchunks, K), jnp.int32),
            pltpu.VMEM((K, D), jnp.float32),
            pltpu.VMEM((K, D), jnp.float32),
            pltpu.SemaphoreType.DMA,
            pltpu.SemaphoreType.DMA,
        ],
    )
    def k(table_hbm, idx_hbm, out_hbm, idx_v, rows0, rows1, sem0, sem1):
        wid = lax.axis_index("s") * info.num_cores + lax.axis_index("c")
        base = wid * gpw
        pltpu.sync_copy(idx_hbm.at[pl.ds(wid * nchunks, nchunks)], idx_v)
        def body(i, _):
            pltpu.async_copy(table_hbm.at[idx_v.at[i]], rows0, sem0).wait()
            pltpu.sync_copy(rows0, out_hbm.at[pl.ds(base + i * K, K)])
            return 0

        lax.fori_loop(0, nchunks, body, 0)
        del rows1, sem1

    return k(table.reshape(table.shape), idx.reshape(G // K, K))


def _sa_body(S, bm, g_ref, fc_ref, w1_ref, b1_ref, w2_ref, b2_ref,
             wr_ref, br_ref, o_ref):
    x = g_ref[0]
    X = jnp.dot(x, w1_ref[...], preferred_element_type=jnp.float32) \
        + b1_ref[...]
    Co = X.shape[-1]
    h = jnp.maximum(X, 0.0)
    h = jnp.dot(h, w2_ref[...],
                preferred_element_type=jnp.float32) + b2_ref[...]
    h = h.reshape(bm, S, Co).max(axis=1)
    r = jnp.dot(fc_ref[0], wr_ref[...],
                preferred_element_type=jnp.float32) + br_ref[...]
    o_ref[0] = jnp.maximum(h + r, 0.0)


def _sa_mlp(g2, fc, W1p, b1, W2, b2, Wr, br):
    B, MS, Dp = g2.shape
    M = fc.shape[1]
    Cin = fc.shape[2]
    S = MS // M
    Co = W1p.shape[1]
    bm = min(M, 256)
    body = functools.partial(_sa_body, S, bm)
    return pl.pallas_call(
        body,
        grid=(B, M // bm),
        in_specs=[
            pl.BlockSpec((1, bm * S, Dp), lambda b, m: (b, m, 0)),
            pl.BlockSpec((1, bm, Cin), lambda b, m: (b, m, 0)),
            pl.BlockSpec((Dp, Co), lambda b, m: (0, 0)),
            pl.BlockSpec((1, Co), lambda b, m: (0, 0)),
            pl.BlockSpec((Co, Co), lambda b, m: (0, 0)),
            pl.BlockSpec((1, Co), lambda b, m: (0, 0)),
            pl.BlockSpec((Cin, Co), lambda b, m: (0, 0)),
            pl.BlockSpec((1, Co), lambda b, m: (0, 0)),
        ],
        out_specs=pl.BlockSpec((1, bm, Co), lambda b, m: (b, m, 0)),
        out_shape=jax.ShapeDtypeStruct((B, M, Co), jnp.float32),
    )(g2, fc, W1p, b1[None], W2, b2[None], Wr, br[None])


def _ln_head(x, w1, b1, lw, lb, w2, b2):
    h = jnp.dot(x, w1, preferred_element_type=jnp.float32) + b1
    mu = jnp.mean(h, axis=-1, keepdims=True)
    var = jnp.mean((h - mu) ** 2, axis=-1, keepdims=True)
    h = (h - mu) * jax.lax.rsqrt(var + 1e-5) * lw + lb
    h = jnp.maximum(h, 0.0)
    return jnp.dot(h, w2, preferred_element_type=jnp.float32) + b2


def _tail_body(B, M, f_ref, info_ref,
               tw_ref, tb_ref, iw_ref, ib_ref,
               a1w_ref, a1b_ref, alw_ref, alb_ref, a2w_ref, a2b_ref,
               o1w_ref, o1b_ref, olw_ref, olb_ref, o2w_ref, o2b_ref,
               feat_ref, pred_ref, off_ref):
    t = jnp.maximum(
        jnp.dot(f_ref[...], tw_ref[...], preferred_element_type=jnp.float32)
        + tb_ref[...], 0.0)
    feats = t.reshape(B, M, 512).max(axis=1)
    feat_ref[...] = feats
    info_f = jnp.dot(info_ref[...], iw_ref[...],
                     preferred_element_type=jnp.float32) + ib_ref[...]
    x = jnp.concatenate([feats, info_f], axis=1)
    pred_ref[...] = _ln_head(x, a1w_ref[...], a1b_ref[...], alw_ref[...],
                             alb_ref[...], a2w_ref[...], a2b_ref[...])
    off_ref[...] = _ln_head(x, o1w_ref[...], o1b_ref[...], olw_ref[...],
                            olb_ref[...], o2w_ref[...], o2b_ref[...])


def _tail_heads(f2d, info, p):
    B = info.shape[0]
    M = f2d.shape[0] // B
    body = functools.partial(_tail_body, B, M)
    full = lambda a: pl.BlockSpec(a.shape, lambda: tuple([0] * a.ndim))
    args = [f2d, info,
            p['tail_W'], p['tail_b'][None], p['info_W'], p['info_b'][None],
            p['a1_W'], p['a1_b'][None], p['a_ln_w'][None], p['a_ln_b'][None],
            p['a2_W'], p['a2_b'][None],
            p['o1_W'], p['o1_b'][None], p['o_ln_w'][None], p['o_ln_b'][None],
            p['o2_W'], p['o2_b'][None]]
    return pl.pallas_call(
        body,
        in_specs=[full(a) for a in args],
        out_specs=[
            pl.BlockSpec((B, 512), lambda: (0, 0)),
            pl.BlockSpec((B, K_CLS), lambda: (0, 0)),
            pl.BlockSpec((B, K_CLS * 3), lambda: (0, 0)),
        ],
        out_shape=[
            jax.ShapeDtypeStruct((B, 512), jnp.float32),
            jax.ShapeDtypeStruct((B, K_CLS), jnp.float32),
            jax.ShapeDtypeStruct((B, K_CLS * 3), jnp.float32),
        ],
    )(*args)


_PAD_BITS = 0x7F800000


def _cumsum_lanes(x, tri):
    N = x.shape[1]
    carry = jnp.zeros((x.shape[0], 1), jnp.float32)
    outs = []
    for k in range(N // 128):
        y = jnp.dot(x[:, k * 128:(k + 1) * 128], tri,
                    preferred_element_type=jnp.float32) + carry
        outs.append(y)
        carry = y[:, 127:128]
    return jnp.concatenate(outs, axis=1)


def _bq_body(r2, N, c_ref, pT_ref, o_ref):
    c = c_ref[0]
    P = pT_ref[0]
    d2 = jnp.zeros((8, N), jnp.float32)
    for dim in range(3):
        diff = P[dim:dim + 1, :] - c[:, dim:dim + 1]
        d2 = d2 + diff * diff
    d2 = jnp.where(d2 <= r2, d2, jnp.inf)
    bits = jax.lax.bitcast_convert_type(d2, jnp.int32)

    acc = jnp.zeros((8, 1), jnp.int32)
    for b in range(30, -1, -1):
        t = acc + (1 << b)
        cnt = jnp.sum((bits < t).astype(jnp.float32), axis=1, keepdims=True)
        acc = jnp.where(cnt < float(NSAMPLE), t, acc)
    T = acc

    ri = jax.lax.broadcasted_iota(jnp.int32, (128, 128), 0)
    ci = jax.lax.broadcasted_iota(jnp.int32, (128, 128), 1)
    tri = (ri <= ci).astype(jnp.float32)

    c_lt = jnp.sum((bits < T).astype(jnp.float32), axis=1, keepdims=True)
    quota = float(NSAMPLE) - c_lt
    eq = bits == T
    eqf = eq.astype(jnp.float32)
    eq_excl = _cumsum_lanes(eqf, tri) - eqf
    in_rad = T < _PAD_BITS
    sel = (bits < T) | (eq & (eq_excl < quota) & in_rad)
    self_f = sel.astype(jnp.float32)
    cs = _cumsum_lanes(self_f, tri)
    cnt_sel = cs[:, N - 1:N]

    cols = []
    for s in range(NSAMPLE):
        cols.append(jnp.sum((cs <= float(s)).astype(jnp.float32),
                            axis=1, keepdims=True))
    idxf = jnp.concatenate(cols, axis=1)

    minb = jnp.min(bits, axis=1, keepdims=True)
    jrow = jax.lax.broadcasted_iota(jnp.int32, (8, N), 1)
    n_c = jnp.min(jnp.where(bits == minb, jrow, N), axis=1, keepdims=True)

    s_iota = jax.lax.broadcasted_iota(jnp.int32, (8, NSAMPLE), 1)
    off = pl.program_id(0) * N
    o_ref[0] = jnp.where(s_iota.astype(jnp.float32) < cnt_sel,
                         idxf.astype(jnp.int32), n_c) + off


def _bq(centers, xyz, radius, nsample):
    B, M, _ = centers.shape
    N = xyz.shape[1]
    xyzT = jnp.swapaxes(xyz, 1, 2)
    body = functools.partial(_bq_body, radius * radius, N)
    return pl.pallas_call(
        body,
        grid=(B, M // 8),
        in_specs=[
            pl.BlockSpec((1, 8, 3), lambda b, m: (b, m, 0)),
            pl.BlockSpec((1, 3, N), lambda b, m: (b, 0, 0)),
        ],
        out_specs=pl.BlockSpec((1, 8, NSAMPLE), lambda b, m: (b, m, 0)),
        out_shape=jax.ShapeDtypeStruct((B, M, NSAMPLE), jnp.int32),
    )(centers, xyzT)


def kernel(points, info, params):
    p = params
    B, N, _ = points.shape
    xyz = points[..., :3]
    f = _stem(points.reshape(B * N, 4), p['stem_W'], p['stem_b'])
    f = f.reshape(B, N, 32)
    radius = BASE_RADIUS
    for i in range(4):
        new_xyz = xyz[:, ::2]
        f_center = f[:, ::2]
        N = xyz.shape[1]
        M = new_xyz.shape[1]
        C = f.shape[2]
        inv_r = 1.0 / radius
        idxg = _bq(new_xyz, xyz, radius, NSAMPLE)
        Dp = ((3 + C + 15) // 16) * 16
        T = jnp.concatenate([xyz, f], axis=-1)
        T = jnp.pad(T, ((0, 0), (0, 0), (0, Dp - (3 + C))))
        G = _sc_gather(T.reshape(B * N, Dp), idxg.reshape(-1))
        g4 = G.reshape(B, M, NSAMPLE, Dp)
        dp = (g4[..., :3] - new_xyz[:, :, None, :]) * inv_r
        G = jnp.concatenate([dp, g4[..., 3:]], axis=-1)
        W1p = jnp.pad(p['sa%d_W1' % i], ((0, Dp - (3 + C)), (0, 0)))
        f = _sa_mlp(G.reshape(B, M * NSAMPLE, Dp), f_center,
                    W1p, p['sa%d_b1' % i],
                    p['sa%d_W2' % i], p['sa%d_b2' % i],
                    p['sa%d_Wr' % i], p['sa%d_br' % i])
        xyz = new_xyz
        radius = radius * RADIUS_SCALING
    features, pred, off = _tail_heads(f.reshape(B * f.shape[1], 512), info, p)
    return (features, pred, off.reshape(-1, K_CLS, 3))


# speedup vs baseline: 6.8455x; 2.8165x over previous
import functools

import jax
import jax.numpy as jnp
from jax import lax
from jax.experimental import pallas as pl
from jax.experimental.pallas import tpu as pltpu
from jax.experimental.pallas import tpu_sc as plsc

K_CLS = 7
NSAMPLE = 32
BASE_RADIUS = 0.15
RADIUS_SCALING = 1.5


def _stem_body(x_ref, w_ref, b_ref, o_ref):
    o_ref[...] = jnp.maximum(
        jnp.dot(x_ref[...], w_ref[...], preferred_element_type=jnp.float32)
        + b_ref[...], 0.0)


def _stem(points2d, W, b):
    R = points2d.shape[0]
    bm = 4096
    return pl.pallas_call(
        _stem_body,
        grid=(R // bm,),
        in_specs=[
            pl.BlockSpec((bm, 4), lambda i: (i, 0)),
            pl.BlockSpec((4, 32), lambda i: (0, 0)),
            pl.BlockSpec((1, 32), lambda i: (0, 0)),
        ],
        out_specs=pl.BlockSpec((bm, 32), lambda i: (i, 0)),
        out_shape=jax.ShapeDtypeStruct((R, 32), jnp.float32),
    )(points2d, W, b[None])


def _sc_gather(table, idx):
    R, D = table.shape
    G = idx.shape[0]
    info = plsc.get_sparse_core_info()
    NW = info.num_cores * info.num_subcores
    gpw = G // NW
    K = 128
    nchunks = gpw // K
    mesh = plsc.VectorSubcoreMesh(core_axis_name="c", subcore_axis_name="s")

    @functools.partial(
        pl.kernel, mesh=mesh,
        compiler_params=pltpu.CompilerParams(use_tc_tiling_on_sc=False),
        out_type=jax.ShapeDtypeStruct((G, D), jnp.float32),
        scratch_types=[
            pltpu.VMEM((nchunks, K), jnp.int32),
            pltpu.VMEM((K, D), jnp.float32),
            pltpu.VMEM((K, D), jnp.float32),
            pltpu.SemaphoreType.DMA,
            pltpu.SemaphoreType.DMA,
        ],
    )
    def k(table_hbm, idx_hbm, out_hbm, idx_v, rows0, rows1, sem0, sem1):
        wid = lax.axis_index("s") * info.num_cores + lax.axis_index("c")
        base = wid * gpw
        pltpu.sync_copy(idx_hbm.at[pl.ds(wid * nchunks, nchunks)], idx_v)
        def body(i, _):
            pltpu.async_copy(table_hbm.at[idx_v.at[i]], rows0, sem0).wait()
            pltpu.sync_copy(rows0, out_hbm.at[pl.ds(base + i * K, K)])
            return 0

        lax.fori_loop(0, nchunks, body, 0)
        del rows1, sem1

    return k(table.reshape(table.shape), idx.reshape(G // K, K))


def _sa_body(S, bm, g_ref, fc_ref, w1_ref, b1_ref, w2_ref, b2_ref,
             wr_ref, br_ref, o_ref):
    x = g_ref[0]
    X = jnp.dot(x, w1_ref[...], preferred_element_type=jnp.float32) \
        + b1_ref[...]
    Co = X.shape[-1]
    h = jnp.maximum(X, 0.0)
    h = jnp.dot(h, w2_ref[...],
                preferred_element_type=jnp.float32) + b2_ref[...]
    h = h.reshape(bm, S, Co).max(axis=1)
    r = jnp.dot(fc_ref[0], wr_ref[...],
                preferred_element_type=jnp.float32) + br_ref[...]
    o_ref[0] = jnp.maximum(h + r, 0.0)


def _sa_mlp(g2, fc, W1p, b1, W2, b2, Wr, br):
    B, MS, Dp = g2.shape
    M = fc.shape[1]
    Cin = fc.shape[2]
    S = MS // M
    Co = W1p.shape[1]
    bm = min(M, 256)
    body = functools.partial(_sa_body, S, bm)
    return pl.pallas_call(
        body,
        grid=(B, M // bm),
        in_specs=[
            pl.BlockSpec((1, bm * S, Dp), lambda b, m: (b, m, 0)),
            pl.BlockSpec((1, bm, Cin), lambda b, m: (b, m, 0)),
            pl.BlockSpec((Dp, Co), lambda b, m: (0, 0)),
            pl.BlockSpec((1, Co), lambda b, m: (0, 0)),
            pl.BlockSpec((Co, Co), lambda b, m: (0, 0)),
            pl.BlockSpec((1, Co), lambda b, m: (0, 0)),
            pl.BlockSpec((Cin, Co), lambda b, m: (0, 0)),
            pl.BlockSpec((1, Co), lambda b, m: (0, 0)),
        ],
        out_specs=pl.BlockSpec((1, bm, Co), lambda b, m: (b, m, 0)),
        out_shape=jax.ShapeDtypeStruct((B, M, Co), jnp.float32),
    )(g2, fc, W1p, b1[None], W2, b2[None], Wr, br[None])


def _ln_head(x, w1, b1, lw, lb, w2, b2):
    h = jnp.dot(x, w1, preferred_element_type=jnp.float32) + b1
    mu = jnp.mean(h, axis=-1, keepdims=True)
    var = jnp.mean((h - mu) ** 2, axis=-1, keepdims=True)
    h = (h - mu) * jax.lax.rsqrt(var + 1e-5) * lw + lb
    h = jnp.maximum(h, 0.0)
    return jnp.dot(h, w2, preferred_element_type=jnp.float32) + b2


def _tail_body(B, M, f_ref, info_ref,
               tw_ref, tb_ref, iw_ref, ib_ref,
               a1w_ref, a1b_ref, alw_ref, alb_ref, a2w_ref, a2b_ref,
               o1w_ref, o1b_ref, olw_ref, olb_ref, o2w_ref, o2b_ref,
               feat_ref, pred_ref, off_ref):
    t = jnp.maximum(
        jnp.dot(f_ref[...], tw_ref[...], preferred_element_type=jnp.float32)
        + tb_ref[...], 0.0)
    feats = t.reshape(B, M, 512).max(axis=1)
    feat_ref[...] = feats
    info_f = jnp.dot(info_ref[...], iw_ref[...],
                     preferred_element_type=jnp.float32) + ib_ref[...]
    x = jnp.concatenate([feats, info_f], axis=1)
    pred_ref[...] = _ln_head(x, a1w_ref[...], a1b_ref[...], alw_ref[...],
                             alb_ref[...], a2w_ref[...], a2b_ref[...])
    off_ref[...] = _ln_head(x, o1w_ref[...], o1b_ref[...], olw_ref[...],
                            olb_ref[...], o2w_ref[...], o2b_ref[...])


def _tail_heads(f2d, info, p):
    B = info.shape[0]
    M = f2d.shape[0] // B
    body = functools.partial(_tail_body, B, M)
    full = lambda a: pl.BlockSpec(a.shape, lambda: tuple([0] * a.ndim))
    args = [f2d, info,
            p['tail_W'], p['tail_b'][None], p['info_W'], p['info_b'][None],
            p['a1_W'], p['a1_b'][None], p['a_ln_w'][None], p['a_ln_b'][None],
            p['a2_W'], p['a2_b'][None],
            p['o1_W'], p['o1_b'][None], p['o_ln_w'][None], p['o_ln_b'][None],
            p['o2_W'], p['o2_b'][None]]
    return pl.pallas_call(
        body,
        in_specs=[full(a) for a in args],
        out_specs=[
            pl.BlockSpec((B, 512), lambda: (0, 0)),
            pl.BlockSpec((B, K_CLS), lambda: (0, 0)),
            pl.BlockSpec((B, K_CLS * 3), lambda: (0, 0)),
        ],
        out_shape=[
            jax.ShapeDtypeStruct((B, 512), jnp.float32),
            jax.ShapeDtypeStruct((B, K_CLS), jnp.float32),
            jax.ShapeDtypeStruct((B, K_CLS * 3), jnp.float32),
        ],
    )(*args)


_PAD_BITS = 0x7F800000


def _cumsum_lanes(x, tri):
    N = x.shape[1]
    carry = jnp.zeros((x.shape[0], 1), jnp.float32)
    outs = []
    for k in range(N // 128):
        y = jnp.dot(x[:, k * 128:(k + 1) * 128], tri,
                    preferred_element_type=jnp.float32) + carry
        outs.append(y)
        carry = y[:, 127:128]
    return jnp.concatenate(outs, axis=1)


def _bq_body(r2, N, BC, c_ref, pT_ref, o_ref):
    c = c_ref[0]
    P = pT_ref[0]
    d2 = jnp.zeros((BC, N), jnp.float32)
    for dim in range(3):
        diff = P[dim:dim + 1, :] - c[:, dim:dim + 1]
        d2 = d2 + diff * diff
    d2 = jnp.where(d2 <= r2, d2, jnp.inf)
    bits = jax.lax.bitcast_convert_type(d2, jnp.int32)

    acc = jnp.zeros((BC, 1), jnp.int32)
    for b in range(30, -1, -1):
        t = acc + (1 << b)
        cnt = jnp.sum((bits < t).astype(jnp.float32), axis=1, keepdims=True)
        acc = jnp.where(cnt < float(NSAMPLE), t, acc)
    T = acc

    ri = jax.lax.broadcasted_iota(jnp.int32, (128, 128), 0)
    ci = jax.lax.broadcasted_iota(jnp.int32, (128, 128), 1)
    tri = (ri <= ci).astype(jnp.float32)

    c_lt = jnp.sum((bits < T).astype(jnp.float32), axis=1, keepdims=True)
    quota = float(NSAMPLE) - c_lt
    eq = bits == T
    eqf = eq.astype(jnp.float32)
    eq_excl = _cumsum_lanes(eqf, tri) - eqf
    in_rad = T < _PAD_BITS
    sel = (bits < T) | (eq & (eq_excl < quota) & in_rad)
    self_f = sel.astype(jnp.float32)
    cs = _cumsum_lanes(self_f, tri)
    cnt_sel = cs[:, N - 1:N]

    cols = []
    for s in range(NSAMPLE):
        cols.append(jnp.sum((cs <= float(s)).astype(jnp.float32),
                            axis=1, keepdims=True))
    idxf = jnp.concatenate(cols, axis=1)

    minb = jnp.min(bits, axis=1, keepdims=True)
    jrow = jax.lax.broadcasted_iota(jnp.int32, (BC, N), 1)
    n_c = jnp.min(jnp.where(bits == minb, jrow, N), axis=1, keepdims=True)

    s_iota = jax.lax.broadcasted_iota(jnp.int32, (BC, NSAMPLE), 1)
    off = pl.program_id(0) * N
    o_ref[0] = jnp.where(s_iota.astype(jnp.float32) < cnt_sel,
                         idxf.astype(jnp.int32), n_c) + off


def _bq(centers, xyz, radius, nsample):
    B, M, _ = centers.shape
    N = xyz.shape[1]
    BC = 32
    xyzT = jnp.swapaxes(xyz, 1, 2)
    body = functools.partial(_bq_body, radius * radius, N, BC)
    return pl.pallas_call(
        body,
        grid=(B, M // BC),
        in_specs=[
            pl.BlockSpec((1, BC, 3), lambda b, m: (b, m, 0)),
            pl.BlockSpec((1, 3, N), lambda b, m: (b, 0, 0)),
        ],
        out_specs=pl.BlockSpec((1, BC, NSAMPLE), lambda b, m: (b, m, 0)),
        out_shape=jax.ShapeDtypeStruct((B, M, NSAMPLE), jnp.int32),
    )(centers, xyzT)


def kernel(points, info, params):
    p = params
    B, N, _ = points.shape
    xyz = points[..., :3]
    f = _stem(points.reshape(B * N, 4), p['stem_W'], p['stem_b'])
    f = f.reshape(B, N, 32)
    radius = BASE_RADIUS
    for i in range(4):
        new_xyz = xyz[:, ::2]
        f_center = f[:, ::2]
        N = xyz.shape[1]
        M = new_xyz.shape[1]
        C = f.shape[2]
        inv_r = 1.0 / radius
        idxg = _bq(new_xyz, xyz, radius, NSAMPLE)
        Dp = ((3 + C + 15) // 16) * 16
        T = jnp.concatenate([xyz, f], axis=-1)
        T = jnp.pad(T, ((0, 0), (0, 0), (0, Dp - (3 + C))))
        G = _sc_gather(T.reshape(B * N, Dp), idxg.reshape(-1))
        g4 = G.reshape(B, M, NSAMPLE, Dp)
        dp = (g4[..., :3] - new_xyz[:, :, None, :]) * inv_r
        G = jnp.concatenate([dp, g4[..., 3:]], axis=-1)
        W1p = jnp.pad(p['sa%d_W1' % i], ((0, Dp - (3 + C)), (0, 0)))
        f = _sa_mlp(G.reshape(B, M * NSAMPLE, Dp), f_center,
                    W1p, p['sa%d_b1' % i],
                    p['sa%d_W2' % i], p['sa%d_b2' % i],
                    p['sa%d_Wr' % i], p['sa%d_br' % i])
        xyz = new_xyz
        radius = radius * RADIUS_SCALING
    features, pred, off = _tail_heads(f.reshape(B * f.shape[1], 512), info, p)
    return (features, pred, off.reshape(-1, K_CLS, 3))


# speedup vs baseline: 9.8491x; 1.4388x over previous
import functools

import jax
import jax.numpy as jnp
from jax import lax
from jax.experimental import pallas as pl
from jax.experimental.pallas import tpu as pltpu
from jax.experimental.pallas import tpu_sc as plsc

K_CLS = 7
NSAMPLE = 32
BASE_RADIUS = 0.15
RADIUS_SCALING = 1.5


def _stem_body(x_ref, w_ref, b_ref, o_ref):
    o_ref[...] = jnp.maximum(
        jnp.dot(x_ref[...], w_ref[...], preferred_element_type=jnp.float32)
        + b_ref[...], 0.0)


def _stem(points2d, W, b):
    R = points2d.shape[0]
    bm = 4096
    return pl.pallas_call(
        _stem_body,
        grid=(R // bm,),
        in_specs=[
            pl.BlockSpec((bm, 4), lambda i: (i, 0)),
            pl.BlockSpec((4, 32), lambda i: (0, 0)),
            pl.BlockSpec((1, 32), lambda i: (0, 0)),
        ],
        out_specs=pl.BlockSpec((bm, 32), lambda i: (i, 0)),
        out_shape=jax.ShapeDtypeStruct((R, 32), jnp.float32),
    )(points2d, W, b[None])


def _sc_gather(table, idx):
    R, D = table.shape
    G = idx.shape[0]
    info = plsc.get_sparse_core_info()
    NW = info.num_cores * info.num_subcores
    gpw = G // NW
    K = 128
    nchunks = gpw // K
    mesh = plsc.VectorSubcoreMesh(core_axis_name="c", subcore_axis_name="s")

    @functools.partial(
        pl.kernel, mesh=mesh,
        compiler_params=pltpu.CompilerParams(use_tc_tiling_on_sc=False),
        out_type=jax.ShapeDtypeStruct((G, D), jnp.float32),
        scratch_types=[
            pltpu.VMEM((nchunks, K), jnp.int32),
            pltpu.VMEM((K, D), jnp.float32),
            pltpu.VMEM((K, D), jnp.float32),
            pltpu.SemaphoreType.DMA,
            pltpu.SemaphoreType.DMA,
        ],
    )
    def k(table_hbm, idx_hbm, out_hbm, idx_v, rows0, rows1, sem0, sem1):
        wid = lax.axis_index("s") * info.num_cores + lax.axis_index("c")
        base = wid * gpw
        pltpu.sync_copy(idx_hbm.at[pl.ds(wid * nchunks, nchunks)], idx_v)
        def body(i, _):
            pltpu.async_copy(table_hbm.at[idx_v.at[i]], rows0, sem0).wait()
            pltpu.sync_copy(rows0, out_hbm.at[pl.ds(base + i * K, K)])
            return 0

        lax.fori_loop(0, nchunks, body, 0)
        del rows1, sem1

    return k(table.reshape(table.shape), idx.reshape(G // K, K))


def _sa_body(S, bm, g_ref, fc_ref, w1_ref, b1_ref, w2_ref, b2_ref,
             wr_ref, br_ref, o_ref):
    x = g_ref[0]
    X = jnp.dot(x, w1_ref[...], preferred_element_type=jnp.float32) \
        + b1_ref[...]
    Co = X.shape[-1]
    h = jnp.maximum(X, 0.0)
    h = jnp.dot(h, w2_ref[...],
                preferred_element_type=jnp.float32) + b2_ref[...]
    h = h.reshape(bm, S, Co).max(axis=1)
    r = jnp.dot(fc_ref[0], wr_ref[...],
                preferred_element_type=jnp.float32) + br_ref[...]
    o_ref[0] = jnp.maximum(h + r, 0.0)


def _sa_mlp(g2, fc, W1p, b1, W2, b2, Wr, br):
    B, MS, Dp = g2.shape
    M = fc.shape[1]
    Cin = fc.shape[2]
    S = MS // M
    Co = W1p.shape[1]
    bm = min(M, 256)
    body = functools.partial(_sa_body, S, bm)
    return pl.pallas_call(
        body,
        grid=(B, M // bm),
        in_specs=[
            pl.BlockSpec((1, bm * S, Dp), lambda b, m: (b, m, 0)),
            pl.BlockSpec((1, bm, Cin), lambda b, m: (b, m, 0)),
            pl.BlockSpec((Dp, Co), lambda b, m: (0, 0)),
            pl.BlockSpec((1, Co), lambda b, m: (0, 0)),
            pl.BlockSpec((Co, Co), lambda b, m: (0, 0)),
            pl.BlockSpec((1, Co), lambda b, m: (0, 0)),
            pl.BlockSpec((Cin, Co), lambda b, m: (0, 0)),
            pl.BlockSpec((1, Co), lambda b, m: (0, 0)),
        ],
        out_specs=pl.BlockSpec((1, bm, Co), lambda b, m: (b, m, 0)),
        out_shape=jax.ShapeDtypeStruct((B, M, Co), jnp.float32),
    )(g2, fc, W1p, b1[None], W2, b2[None], Wr, br[None])


def _ln_head(x, w1, b1, lw, lb, w2, b2):
    h = jnp.dot(x, w1, preferred_element_type=jnp.float32) + b1
    mu = jnp.mean(h, axis=-1, keepdims=True)
    var = jnp.mean((h - mu) ** 2, axis=-1, keepdims=True)
    h = (h - mu) * jax.lax.rsqrt(var + 1e-5) * lw + lb
    h = jnp.maximum(h, 0.0)
    return jnp.dot(h, w2, preferred_element_type=jnp.float32) + b2


def _tail_body(B, M, f_ref, info_ref,
               tw_ref, tb_ref, iw_ref, ib_ref,
               a1w_ref, a1b_ref, alw_ref, alb_ref, a2w_ref, a2b_ref,
               o1w_ref, o1b_ref, olw_ref, olb_ref, o2w_ref, o2b_ref,
               feat_ref, pred_ref, off_ref):
    t = jnp.maximum(
        jnp.dot(f_ref[...], tw_ref[...], preferred_element_type=jnp.float32)
        + tb_ref[...], 0.0)
    feats = t.reshape(B, M, 512).max(axis=1)
    feat_ref[...] = feats
    info_f = jnp.dot(info_ref[...], iw_ref[...],
                     preferred_element_type=jnp.float32) + ib_ref[...]
    x = jnp.concatenate([feats, info_f], axis=1)
    pred_ref[...] = _ln_head(x, a1w_ref[...], a1b_ref[...], alw_ref[...],
                             alb_ref[...], a2w_ref[...], a2b_ref[...])
    off_ref[...] = _ln_head(x, o1w_ref[...], o1b_ref[...], olw_ref[...],
                            olb_ref[...], o2w_ref[...], o2b_ref[...])


def _tail_heads(f2d, info, p):
    B = info.shape[0]
    M = f2d.shape[0] // B
    body = functools.partial(_tail_body, B, M)
    full = lambda a: pl.BlockSpec(a.shape, lambda: tuple([0] * a.ndim))
    args = [f2d, info,
            p['tail_W'], p['tail_b'][None], p['info_W'], p['info_b'][None],
            p['a1_W'], p['a1_b'][None], p['a_ln_w'][None], p['a_ln_b'][None],
            p['a2_W'], p['a2_b'][None],
            p['o1_W'], p['o1_b'][None], p['o_ln_w'][None], p['o_ln_b'][None],
            p['o2_W'], p['o2_b'][None]]
    return pl.pallas_call(
        body,
        in_specs=[full(a) for a in args],
        out_specs=[
            pl.BlockSpec((B, 512), lambda: (0, 0)),
            pl.BlockSpec((B, K_CLS), lambda: (0, 0)),
            pl.BlockSpec((B, K_CLS * 3), lambda: (0, 0)),
        ],
        out_shape=[
            jax.ShapeDtypeStruct((B, 512), jnp.float32),
            jax.ShapeDtypeStruct((B, K_CLS), jnp.float32),
            jax.ShapeDtypeStruct((B, K_CLS * 3), jnp.float32),
        ],
    )(*args)


_PAD_BITS = 0x7F800000


def _cumsum_lanes(x, tri):
    N = x.shape[1]
    carry = jnp.zeros((x.shape[0], 1), jnp.float32)
    outs = []
    for k in range(N // 128):
        y = jnp.dot(x[:, k * 128:(k + 1) * 128], tri,
                    preferred_element_type=jnp.float32) + carry
        outs.append(y)
        carry = y[:, 127:128]
    return jnp.concatenate(outs, axis=1)


def _bq_body(r2, N, BC, c_ref, pT_ref, o_ref):
    c = c_ref[0]
    P = pT_ref[0]
    d2 = jnp.zeros((BC, N), jnp.float32)
    for dim in range(3):
        diff = P[dim:dim + 1, :] - c[:, dim:dim + 1]
        d2 = d2 + diff * diff
    d2 = jnp.where(d2 <= r2, d2, jnp.inf)
    bits = jax.lax.bitcast_convert_type(d2, jnp.int32)

    acc = jnp.zeros((BC, 1), jnp.int32)
    for b in range(30, -1, -1):
        t = acc + (1 << b)
        cnt = jnp.sum((bits < t).astype(jnp.float32), axis=1, keepdims=True)
        acc = jnp.where(cnt < float(NSAMPLE), t, acc)
    T = acc

    ri = jax.lax.broadcasted_iota(jnp.int32, (128, 128), 0)
    ci = jax.lax.broadcasted_iota(jnp.int32, (128, 128), 1)
    tri = (ri <= ci).astype(jnp.float32)

    c_lt = jnp.sum((bits < T).astype(jnp.float32), axis=1, keepdims=True)
    quota = float(NSAMPLE) - c_lt
    eq = bits == T
    eqf = eq.astype(jnp.float32)
    eq_excl = _cumsum_lanes(eqf, tri) - eqf
    in_rad = T < _PAD_BITS
    sel = (bits < T) | (eq & (eq_excl < quota) & in_rad)
    self_f = sel.astype(jnp.float32)
    cs = _cumsum_lanes(self_f, tri)
    cnt_sel = cs[:, N - 1:N]

    cols = []
    for s in range(NSAMPLE):
        cols.append(jnp.sum((cs <= float(s)).astype(jnp.float32),
                            axis=1, keepdims=True))
    idxf = jnp.concatenate(cols, axis=1)

    minb = jnp.min(bits, axis=1, keepdims=True)
    jrow = jax.lax.broadcasted_iota(jnp.int32, (BC, N), 1)
    n_c = jnp.min(jnp.where(bits == minb, jrow, N), axis=1, keepdims=True)

    s_iota = jax.lax.broadcasted_iota(jnp.int32, (BC, NSAMPLE), 1)
    off = pl.program_id(0) * N
    o_ref[0] = jnp.where(s_iota.astype(jnp.float32) < cnt_sel,
                         idxf.astype(jnp.int32), n_c) + off


def _bq(centers, xyz, radius, nsample):
    B, M, _ = centers.shape
    N = xyz.shape[1]
    BC = 128
    xyzT = jnp.swapaxes(xyz, 1, 2)
    body = functools.partial(_bq_body, radius * radius, N, BC)
    return pl.pallas_call(
        body,
        grid=(B, M // BC),
        in_specs=[
            pl.BlockSpec((1, BC, 3), lambda b, m: (b, m, 0)),
            pl.BlockSpec((1, 3, N), lambda b, m: (b, 0, 0)),
        ],
        out_specs=pl.BlockSpec((1, BC, NSAMPLE), lambda b, m: (b, m, 0)),
        out_shape=jax.ShapeDtypeStruct((B, M, NSAMPLE), jnp.int32),
    )(centers, xyzT)


def kernel(points, info, params):
    p = params
    B, N, _ = points.shape
    xyz = points[..., :3]
    f = _stem(points.reshape(B * N, 4), p['stem_W'], p['stem_b'])
    f = f.reshape(B, N, 32)
    radius = BASE_RADIUS
    for i in range(4):
        new_xyz = xyz[:, ::2]
        f_center = f[:, ::2]
        N = xyz.shape[1]
        M = new_xyz.shape[1]
        C = f.shape[2]
        inv_r = 1.0 / radius
        idxg = _bq(new_xyz, xyz, radius, NSAMPLE)
        Dp = ((3 + C + 15) // 16) * 16
        T = jnp.concatenate([xyz, f], axis=-1)
        T = jnp.pad(T, ((0, 0), (0, 0), (0, Dp - (3 + C))))
        G = _sc_gather(T.reshape(B * N, Dp), idxg.reshape(-1))
        g4 = G.reshape(B, M, NSAMPLE, Dp)
        dp = (g4[..., :3] - new_xyz[:, :, None, :]) * inv_r
        G = jnp.concatenate([dp, g4[..., 3:]], axis=-1)
        W1p = jnp.pad(p['sa%d_W1' % i], ((0, Dp - (3 + C)), (0, 0)))
        f = _sa_mlp(G.reshape(B, M * NSAMPLE, Dp), f_center,
                    W1p, p['sa%d_b1' % i],
                    p['sa%d_W2' % i], p['sa%d_b2' % i],
                    p['sa%d_Wr' % i], p['sa%d_br' % i])
        xyz = new_xyz
        radius = radius * RADIUS_SCALING
    features, pred, off = _tail_heads(f.reshape(B * f.shape[1], 512), info, p)
    return (features, pred, off.reshape(-1, K_CLS, 3))


# speedup vs baseline: 10.3172x; 1.0475x over previous
import functools

import jax
import jax.numpy as jnp
from jax import lax
from jax.experimental import pallas as pl
from jax.experimental.pallas import tpu as pltpu
from jax.experimental.pallas import tpu_sc as plsc

K_CLS = 7
NSAMPLE = 32
BASE_RADIUS = 0.15
RADIUS_SCALING = 1.5


def _stem_body(x_ref, w_ref, b_ref, o_ref):
    o_ref[...] = jnp.maximum(
        jnp.dot(x_ref[...], w_ref[...], preferred_element_type=jnp.float32)
        + b_ref[...], 0.0)


def _stem(points2d, W, b):
    R = points2d.shape[0]
    bm = 4096
    return pl.pallas_call(
        _stem_body,
        grid=(R // bm,),
        in_specs=[
            pl.BlockSpec((bm, 4), lambda i: (i, 0)),
            pl.BlockSpec((4, 32), lambda i: (0, 0)),
            pl.BlockSpec((1, 32), lambda i: (0, 0)),
        ],
        out_specs=pl.BlockSpec((bm, 32), lambda i: (i, 0)),
        out_shape=jax.ShapeDtypeStruct((R, 32), jnp.float32),
    )(points2d, W, b[None])


def _sc_gather(table, idx):
    R, D = table.shape
    G = idx.shape[0]
    info = plsc.get_sparse_core_info()
    NW = info.num_cores * info.num_subcores
    gpw = G // NW
    K = 128
    nchunks = gpw // K
    mesh = plsc.VectorSubcoreMesh(core_axis_name="c", subcore_axis_name="s")

    @functools.partial(
        pl.kernel, mesh=mesh,
        compiler_params=pltpu.CompilerParams(use_tc_tiling_on_sc=False),
        out_type=jax.ShapeDtypeStruct((G, D), jnp.float32),
        scratch_types=[
            pltpu.VMEM((nchunks, K), jnp.int32),
            pltpu.VMEM((K, D), jnp.float32),
            pltpu.VMEM((K, D), jnp.float32),
            pltpu.SemaphoreType.DMA,
            pltpu.SemaphoreType.DMA,
        ],
    )
    def k(table_hbm, idx_hbm, out_hbm, idx_v, rows0, rows1, sem0, sem1):
        wid = lax.axis_index("s") * info.num_cores + lax.axis_index("c")
        base = wid * gpw
        pltpu.sync_copy(idx_hbm.at[pl.ds(wid * nchunks, nchunks)], idx_v)
        def body(i, _):
            pltpu.async_copy(table_hbm.at[idx_v.at[i]], rows0, sem0).wait()
            pltpu.sync_copy(rows0, out_hbm.at[pl.ds(base + i * K, K)])
            return 0

        lax.fori_loop(0, nchunks, body, 0)
        del rows1, sem1

    return k(table.reshape(table.shape), idx.reshape(G // K, K))


def _sa_body(S, bm, g_ref, fc_ref, w1_ref, b1_ref, w2_ref, b2_ref,
             wr_ref, br_ref, o_ref):
    x = g_ref[0]
    X = jnp.dot(x, w1_ref[...], preferred_element_type=jnp.float32) \
        + b1_ref[...]
    Co = X.shape[-1]
    h = jnp.maximum(X, 0.0)
    h = jnp.dot(h, w2_ref[...],
                preferred_element_type=jnp.float32) + b2_ref[...]
    h = h.reshape(bm, S, Co).max(axis=1)
    r = jnp.dot(fc_ref[0], wr_ref[...],
                preferred_element_type=jnp.float32) + br_ref[...]
    o_ref[0] = jnp.maximum(h + r, 0.0)


def _sa_mlp(g2, fc, W1p, b1, W2, b2, Wr, br):
    B, MS, Dp = g2.shape
    M = fc.shape[1]
    Cin = fc.shape[2]
    S = MS // M
    Co = W1p.shape[1]
    bm = min(M, 256)
    body = functools.partial(_sa_body, S, bm)
    return pl.pallas_call(
        body,
        grid=(B, M // bm),
        in_specs=[
            pl.BlockSpec((1, bm * S, Dp), lambda b, m: (b, m, 0)),
            pl.BlockSpec((1, bm, Cin), lambda b, m: (b, m, 0)),
            pl.BlockSpec((Dp, Co), lambda b, m: (0, 0)),
            pl.BlockSpec((1, Co), lambda b, m: (0, 0)),
            pl.BlockSpec((Co, Co), lambda b, m: (0, 0)),
            pl.BlockSpec((1, Co), lambda b, m: (0, 0)),
            pl.BlockSpec((Cin, Co), lambda b, m: (0, 0)),
            pl.BlockSpec((1, Co), lambda b, m: (0, 0)),
        ],
        out_specs=pl.BlockSpec((1, bm, Co), lambda b, m: (b, m, 0)),
        out_shape=jax.ShapeDtypeStruct((B, M, Co), jnp.float32),
    )(g2, fc, W1p, b1[None], W2, b2[None], Wr, br[None])


def _ln_head(x, w1, b1, lw, lb, w2, b2):
    h = jnp.dot(x, w1, preferred_element_type=jnp.float32) + b1
    mu = jnp.mean(h, axis=-1, keepdims=True)
    var = jnp.mean((h - mu) ** 2, axis=-1, keepdims=True)
    h = (h - mu) * jax.lax.rsqrt(var + 1e-5) * lw + lb
    h = jnp.maximum(h, 0.0)
    return jnp.dot(h, w2, preferred_element_type=jnp.float32) + b2


def _tail_body(B, M, f_ref, info_ref,
               tw_ref, tb_ref, iw_ref, ib_ref,
               a1w_ref, a1b_ref, alw_ref, alb_ref, a2w_ref, a2b_ref,
               o1w_ref, o1b_ref, olw_ref, olb_ref, o2w_ref, o2b_ref,
               feat_ref, pred_ref, off_ref):
    t = jnp.maximum(
        jnp.dot(f_ref[...], tw_ref[...], preferred_element_type=jnp.float32)
        + tb_ref[...], 0.0)
    feats = t.reshape(B, M, 512).max(axis=1)
    feat_ref[...] = feats
    info_f = jnp.dot(info_ref[...], iw_ref[...],
                     preferred_element_type=jnp.float32) + ib_ref[...]
    x = jnp.concatenate([feats, info_f], axis=1)
    pred_ref[...] = _ln_head(x, a1w_ref[...], a1b_ref[...], alw_ref[...],
                             alb_ref[...], a2w_ref[...], a2b_ref[...])
    off_ref[...] = _ln_head(x, o1w_ref[...], o1b_ref[...], olw_ref[...],
                            olb_ref[...], o2w_ref[...], o2b_ref[...])


def _tail_heads(f2d, info, p):
    B = info.shape[0]
    M = f2d.shape[0] // B
    body = functools.partial(_tail_body, B, M)
    full = lambda a: pl.BlockSpec(a.shape, lambda: tuple([0] * a.ndim))
    args = [f2d, info,
            p['tail_W'], p['tail_b'][None], p['info_W'], p['info_b'][None],
            p['a1_W'], p['a1_b'][None], p['a_ln_w'][None], p['a_ln_b'][None],
            p['a2_W'], p['a2_b'][None],
            p['o1_W'], p['o1_b'][None], p['o_ln_w'][None], p['o_ln_b'][None],
            p['o2_W'], p['o2_b'][None]]
    return pl.pallas_call(
        body,
        in_specs=[full(a) for a in args],
        out_specs=[
            pl.BlockSpec((B, 512), lambda: (0, 0)),
            pl.BlockSpec((B, K_CLS), lambda: (0, 0)),
            pl.BlockSpec((B, K_CLS * 3), lambda: (0, 0)),
        ],
        out_shape=[
            jax.ShapeDtypeStruct((B, 512), jnp.float32),
            jax.ShapeDtypeStruct((B, K_CLS), jnp.float32),
            jax.ShapeDtypeStruct((B, K_CLS * 3), jnp.float32),
        ],
    )(*args)


_PAD_BITS = 0x7F800000


def _cumsum_lanes(x, tri):
    N = x.shape[1]
    carry = jnp.zeros((x.shape[0], 1), jnp.float32)
    outs = []
    for k in range(N // 128):
        y = jnp.dot(x[:, k * 128:(k + 1) * 128], tri,
                    preferred_element_type=jnp.float32) + carry
        outs.append(y)
        carry = y[:, 127:128]
    return jnp.concatenate(outs, axis=1)


def _bq_body(r2, N, BC, c_ref, pT_ref, o_ref):
    c = c_ref[0]
    P = pT_ref[0]
    d2 = jnp.zeros((BC, N), jnp.float32)
    for dim in range(3):
        diff = P[dim:dim + 1, :] - c[:, dim:dim + 1]
        d2 = d2 + diff * diff
    d2 = jnp.where(d2 <= r2, d2, jnp.inf)
    bits = jax.lax.bitcast_convert_type(d2, jnp.int32)

    acc = jnp.zeros((BC, 1), jnp.int32)
    for b in range(30, -1, -1):
        t = acc + (1 << b)
        cnt = jnp.sum((bits < t).astype(jnp.float32), axis=1, keepdims=True)
        acc = jnp.where(cnt < float(NSAMPLE), t, acc)
    T = acc

    ri = jax.lax.broadcasted_iota(jnp.int32, (128, 128), 0)
    ci = jax.lax.broadcasted_iota(jnp.int32, (128, 128), 1)
    tri = (ri <= ci).astype(jnp.float32)

    c_lt = jnp.sum((bits < T).astype(jnp.float32), axis=1, keepdims=True)
    quota = float(NSAMPLE) - c_lt
    eq = bits == T
    eqf = eq.astype(jnp.float32)
    eq_excl = _cumsum_lanes(eqf, tri) - eqf
    in_rad = T < _PAD_BITS
    sel = (bits < T) | (eq & (eq_excl < quota) & in_rad)
    self_f = sel.astype(jnp.float32)
    cs = _cumsum_lanes(self_f, tri)
    cnt_sel = cs[:, N - 1:N]

    cols = []
    for s in range(NSAMPLE):
        cols.append(jnp.sum((cs <= float(s)).astype(jnp.float32),
                            axis=1, keepdims=True))
    idxf = jnp.concatenate(cols, axis=1)

    minb = jnp.min(bits, axis=1, keepdims=True)
    jrow = jax.lax.broadcasted_iota(jnp.int32, (BC, N), 1)
    n_c = jnp.min(jnp.where(bits == minb, jrow, N), axis=1, keepdims=True)

    s_iota = jax.lax.broadcasted_iota(jnp.int32, (BC, NSAMPLE), 1)
    off = pl.program_id(0) * N
    o_ref[0] = jnp.where(s_iota.astype(jnp.float32) < cnt_sel,
                         idxf.astype(jnp.int32), n_c) + off


def _bq(centers, xyz, radius, nsample):
    B, M, _ = centers.shape
    N = xyz.shape[1]
    BC = min(M, 256)
    xyzT = jnp.swapaxes(xyz, 1, 2)
    body = functools.partial(_bq_body, radius * radius, N, BC)
    return pl.pallas_call(
        body,
        grid=(B, M // BC),
        in_specs=[
            pl.BlockSpec((1, BC, 3), lambda b, m: (b, m, 0)),
            pl.BlockSpec((1, 3, N), lambda b, m: (b, 0, 0)),
        ],
        out_specs=pl.BlockSpec((1, BC, NSAMPLE), lambda b, m: (b, m, 0)),
        out_shape=jax.ShapeDtypeStruct((B, M, NSAMPLE), jnp.int32),
    )(centers, xyzT)


def kernel(points, info, params):
    p = params
    B, N, _ = points.shape
    xyz = points[..., :3]
    f = _stem(points.reshape(B * N, 4), p['stem_W'], p['stem_b'])
    f = f.reshape(B, N, 32)
    radius = BASE_RADIUS
    for i in range(4):
        new_xyz = xyz[:, ::2]
        f_center = f[:, ::2]
        N = xyz.shape[1]
        M = new_xyz.shape[1]
        C = f.shape[2]
        inv_r = 1.0 / radius
        idxg = _bq(new_xyz, xyz, radius, NSAMPLE)
        Dp = ((3 + C + 15) // 16) * 16
        T = jnp.concatenate([xyz, f], axis=-1)
        T = jnp.pad(T, ((0, 0), (0, 0), (0, Dp - (3 + C))))
        G = _sc_gather(T.reshape(B * N, Dp), idxg.reshape(-1))
        g4 = G.reshape(B, M, NSAMPLE, Dp)
        dp = (g4[..., :3] - new_xyz[:, :, None, :]) * inv_r
        G = jnp.concatenate([dp, g4[..., 3:]], axis=-1)
        W1p = jnp.pad(p['sa%d_W1' % i], ((0, Dp - (3 + C)), (0, 0)))
        f = _sa_mlp(G.reshape(B, M * NSAMPLE, Dp), f_center,
                    W1p, p['sa%d_b1' % i],
                    p['sa%d_W2' % i], p['sa%d_b2' % i],
                    p['sa%d_Wr' % i], p['sa%d_br' % i])
        xyz = new_xyz
        radius = radius * RADIUS_SCALING
    features, pred, off = _tail_heads(f.reshape(B * f.shape[1], 512), info, p)
    return (features, pred, off.reshape(-1, K_CLS, 3))


# speedup vs baseline: 10.5187x; 1.0195x over previous
import functools

import jax
import jax.numpy as jnp
from jax import lax
from jax.experimental import pallas as pl
from jax.experimental.pallas import tpu as pltpu
from jax.experimental.pallas import tpu_sc as plsc

K_CLS = 7
NSAMPLE = 32
BASE_RADIUS = 0.15
RADIUS_SCALING = 1.5


def _stem_body(x_ref, w_ref, b_ref, o_ref):
    o_ref[...] = jnp.maximum(
        jnp.dot(x_ref[...], w_ref[...], preferred_element_type=jnp.float32)
        + b_ref[...], 0.0)


def _stem(points2d, W, b):
    R = points2d.shape[0]
    bm = 4096
    return pl.pallas_call(
        _stem_body,
        grid=(R // bm,),
        in_specs=[
            pl.BlockSpec((bm, 4), lambda i: (i, 0)),
            pl.BlockSpec((4, 32), lambda i: (0, 0)),
            pl.BlockSpec((1, 32), lambda i: (0, 0)),
        ],
        out_specs=pl.BlockSpec((bm, 32), lambda i: (i, 0)),
        out_shape=jax.ShapeDtypeStruct((R, 32), jnp.float32),
    )(points2d, W, b[None])


def _sc_gather(table, idx):
    R, D = table.shape
    G = idx.shape[0]
    info = plsc.get_sparse_core_info()
    NW = info.num_cores * info.num_subcores
    gpw = G // NW
    K = 128
    nchunks = gpw // K
    mesh = plsc.VectorSubcoreMesh(core_axis_name="c", subcore_axis_name="s")

    @functools.partial(
        pl.kernel, mesh=mesh,
        compiler_params=pltpu.CompilerParams(use_tc_tiling_on_sc=False),
        out_type=jax.ShapeDtypeStruct((G, D), jnp.float32),
        scratch_types=[
            pltpu.VMEM((nchunks, K), jnp.int32),
            pltpu.VMEM((K, D), jnp.float32),
            pltpu.VMEM((K, D), jnp.float32),
            pltpu.SemaphoreType.DMA,
            pltpu.SemaphoreType.DMA,
        ],
    )
    def k(table_hbm, idx_hbm, out_hbm, idx_v, rows0, rows1, sem0, sem1):
        wid = lax.axis_index("s") * info.num_cores + lax.axis_index("c")
        base = wid * gpw
        pltpu.sync_copy(idx_hbm.at[pl.ds(wid * nchunks, nchunks)], idx_v)
        pltpu.async_copy(table_hbm.at[idx_v.at[0]], rows0, sem0)

        def body(g, _):
            i0 = 2 * g
            pltpu.async_copy(table_hbm.at[idx_v.at[i0 + 1]], rows1, sem1)
            pltpu.make_async_copy(table_hbm.at[idx_v.at[i0]],
                                  rows0, sem0).wait()
            pltpu.sync_copy(rows0, out_hbm.at[pl.ds(base + i0 * K, K)])

            @pl.when(i0 + 2 < nchunks)
            def _():
                pltpu.async_copy(table_hbm.at[idx_v.at[i0 + 2]],
                                 rows0, sem0)

            pltpu.make_async_copy(table_hbm.at[idx_v.at[i0 + 1]],
                                  rows1, sem1).wait()
            pltpu.sync_copy(rows1, out_hbm.at[pl.ds(base + (i0 + 1) * K, K)])
            return 0

        lax.fori_loop(0, nchunks // 2, body, 0)

    return k(table.reshape(table.shape), idx.reshape(G // K, K))


def _sa_body(S, bm, g_ref, fc_ref, w1_ref, b1_ref, w2_ref, b2_ref,
             wr_ref, br_ref, o_ref):
    x = g_ref[0]
    X = jnp.dot(x, w1_ref[...], preferred_element_type=jnp.float32) \
        + b1_ref[...]
    Co = X.shape[-1]
    h = jnp.maximum(X, 0.0)
    h = jnp.dot(h, w2_ref[...],
                preferred_element_type=jnp.float32) + b2_ref[...]
    h = h.reshape(bm, S, Co).max(axis=1)
    r = jnp.dot(fc_ref[0], wr_ref[...],
                preferred_element_type=jnp.float32) + br_ref[...]
    o_ref[0] = jnp.maximum(h + r, 0.0)


def _sa_mlp(g2, fc, W1p, b1, W2, b2, Wr, br):
    B, MS, Dp = g2.shape
    M = fc.shape[1]
    Cin = fc.shape[2]
    S = MS // M
    Co = W1p.shape[1]
    bm = min(M, 256)
    body = functools.partial(_sa_body, S, bm)
    return pl.pallas_call(
        body,
        grid=(B, M // bm),
        in_specs=[
            pl.BlockSpec((1, bm * S, Dp), lambda b, m: (b, m, 0)),
            pl.BlockSpec((1, bm, Cin), lambda b, m: (b, m, 0)),
            pl.BlockSpec((Dp, Co), lambda b, m: (0, 0)),
            pl.BlockSpec((1, Co), lambda b, m: (0, 0)),
            pl.BlockSpec((Co, Co), lambda b, m: (0, 0)),
            pl.BlockSpec((1, Co), lambda b, m: (0, 0)),
            pl.BlockSpec((Cin, Co), lambda b, m: (0, 0)),
            pl.BlockSpec((1, Co), lambda b, m: (0, 0)),
        ],
        out_specs=pl.BlockSpec((1, bm, Co), lambda b, m: (b, m, 0)),
        out_shape=jax.ShapeDtypeStruct((B, M, Co), jnp.float32),
    )(g2, fc, W1p, b1[None], W2, b2[None], Wr, br[None])


def _ln_head(x, w1, b1, lw, lb, w2, b2):
    h = jnp.dot(x, w1, preferred_element_type=jnp.float32) + b1
    mu = jnp.mean(h, axis=-1, keepdims=True)
    var = jnp.mean((h - mu) ** 2, axis=-1, keepdims=True)
    h = (h - mu) * jax.lax.rsqrt(var + 1e-5) * lw + lb
    h = jnp.maximum(h, 0.0)
    return jnp.dot(h, w2, preferred_element_type=jnp.float32) + b2


def _tail_body(B, M, f_ref, info_ref,
               tw_ref, tb_ref, iw_ref, ib_ref,
               a1w_ref, a1b_ref, alw_ref, alb_ref, a2w_ref, a2b_ref,
               o1w_ref, o1b_ref, olw_ref, olb_ref, o2w_ref, o2b_ref,
               feat_ref, pred_ref, off_ref):
    t = jnp.maximum(
        jnp.dot(f_ref[...], tw_ref[...], preferred_element_type=jnp.float32)
        + tb_ref[...], 0.0)
    feats = t.reshape(B, M, 512).max(axis=1)
    feat_ref[...] = feats
    info_f = jnp.dot(info_ref[...], iw_ref[...],
                     preferred_element_type=jnp.float32) + ib_ref[...]
    x = jnp.concatenate([feats, info_f], axis=1)
    pred_ref[...] = _ln_head(x, a1w_ref[...], a1b_ref[...], alw_ref[...],
                             alb_ref[...], a2w_ref[...], a2b_ref[...])
    off_ref[...] = _ln_head(x, o1w_ref[...], o1b_ref[...], olw_ref[...],
                            olb_ref[...], o2w_ref[...], o2b_ref[...])


def _tail_heads(f2d, info, p):
    B = info.shape[0]
    M = f2d.shape[0] // B
    body = functools.partial(_tail_body, B, M)
    full = lambda a: pl.BlockSpec(a.shape, lambda: tuple([0] * a.ndim))
    args = [f2d, info,
            p['tail_W'], p['tail_b'][None], p['info_W'], p['info_b'][None],
            p['a1_W'], p['a1_b'][None], p['a_ln_w'][None], p['a_ln_b'][None],
            p['a2_W'], p['a2_b'][None],
            p['o1_W'], p['o1_b'][None], p['o_ln_w'][None], p['o_ln_b'][None],
            p['o2_W'], p['o2_b'][None]]
    return pl.pallas_call(
        body,
        in_specs=[full(a) for a in args],
        out_specs=[
            pl.BlockSpec((B, 512), lambda: (0, 0)),
            pl.BlockSpec((B, K_CLS), lambda: (0, 0)),
            pl.BlockSpec((B, K_CLS * 3), lambda: (0, 0)),
        ],
        out_shape=[
            jax.ShapeDtypeStruct((B, 512), jnp.float32),
            jax.ShapeDtypeStruct((B, K_CLS), jnp.float32),
            jax.ShapeDtypeStruct((B, K_CLS * 3), jnp.float32),
        ],
    )(*args)


_PAD_BITS = 0x7F800000


def _cumsum_lanes(x, tri):
    N = x.shape[1]
    carry = jnp.zeros((x.shape[0], 1), jnp.float32)
    outs = []
    for k in range(N // 128):
        y = jnp.dot(x[:, k * 128:(k + 1) * 128], tri,
                    preferred_element_type=jnp.float32) + carry
        outs.append(y)
        carry = y[:, 127:128]
    return jnp.concatenate(outs, axis=1)


def _bq_body(r2, N, BC, c_ref, pT_ref, o_ref):
    c = c_ref[0]
    P = pT_ref[0]
    d2 = jnp.zeros((BC, N), jnp.float32)
    for dim in range(3):
        diff = P[dim:dim + 1, :] - c[:, dim:dim + 1]
        d2 = d2 + diff * diff
    d2 = jnp.where(d2 <= r2, d2, jnp.inf)
    bits = jax.lax.bitcast_convert_type(d2, jnp.int32)

    acc = jnp.zeros((BC, 1), jnp.int32)
    for b in range(30, -1, -1):
        t = acc + (1 << b)
        cnt = jnp.sum((bits < t).astype(jnp.float32), axis=1, keepdims=True)
        acc = jnp.where(cnt < float(NSAMPLE), t, acc)
    T = acc

    ri = jax.lax.broadcasted_iota(jnp.int32, (128, 128), 0)
    ci = jax.lax.broadcasted_iota(jnp.int32, (128, 128), 1)
    tri = (ri <= ci).astype(jnp.float32)

    c_lt = jnp.sum((bits < T).astype(jnp.float32), axis=1, keepdims=True)
    quota = float(NSAMPLE) - c_lt
    eq = bits == T
    eqf = eq.astype(jnp.float32)
    eq_excl = _cumsum_lanes(eqf, tri) - eqf
    in_rad = T < _PAD_BITS
    sel = (bits < T) | (eq & (eq_excl < quota) & in_rad)
    self_f = sel.astype(jnp.float32)
    cs = _cumsum_lanes(self_f, tri)
    cnt_sel = cs[:, N - 1:N]

    cols = []
    for s in range(NSAMPLE):
        cols.append(jnp.sum((cs <= float(s)).astype(jnp.float32),
                            axis=1, keepdims=True))
    idxf = jnp.concatenate(cols, axis=1)

    minb = jnp.min(bits, axis=1, keepdims=True)
    jrow = jax.lax.broadcasted_iota(jnp.int32, (BC, N), 1)
    n_c = jnp.min(jnp.where(bits == minb, jrow, N), axis=1, keepdims=True)

    s_iota = jax.lax.broadcasted_iota(jnp.int32, (BC, NSAMPLE), 1)
    off = pl.program_id(0) * N
    o_ref[0] = jnp.where(s_iota.astype(jnp.float32) < cnt_sel,
                         idxf.astype(jnp.int32), n_c) + off


def _bq(centers, xyz, radius, nsample):
    B, M, _ = centers.shape
    N = xyz.shape[1]
    BC = min(M, 256)
    xyzT = jnp.swapaxes(xyz, 1, 2)
    body = functools.partial(_bq_body, radius * radius, N, BC)
    return pl.pallas_call(
        body,
        grid=(B, M // BC),
        in_specs=[
            pl.BlockSpec((1, BC, 3), lambda b, m: (b, m, 0)),
            pl.BlockSpec((1, 3, N), lambda b, m: (b, 0, 0)),
        ],
        out_specs=pl.BlockSpec((1, BC, NSAMPLE), lambda b, m: (b, m, 0)),
        out_shape=jax.ShapeDtypeStruct((B, M, NSAMPLE), jnp.int32),
    )(centers, xyzT)


def kernel(points, info, params):
    p = params
    B, N, _ = points.shape
    xyz = points[..., :3]
    f = _stem(points.reshape(B * N, 4), p['stem_W'], p['stem_b'])
    f = f.reshape(B, N, 32)
    radius = BASE_RADIUS
    for i in range(4):
        new_xyz = xyz[:, ::2]
        f_center = f[:, ::2]
        N = xyz.shape[1]
        M = new_xyz.shape[1]
        C = f.shape[2]
        inv_r = 1.0 / radius
        idxg = _bq(new_xyz, xyz, radius, NSAMPLE)
        Dp = ((3 + C + 15) // 16) * 16
        T = jnp.concatenate([xyz, f], axis=-1)
        T = jnp.pad(T, ((0, 0), (0, 0), (0, Dp - (3 + C))))
        G = _sc_gather(T.reshape(B * N, Dp), idxg.reshape(-1))
        g4 = G.reshape(B, M, NSAMPLE, Dp)
        dp = (g4[..., :3] - new_xyz[:, :, None, :]) * inv_r
        G = jnp.concatenate([dp, g4[..., 3:]], axis=-1)
        W1p = jnp.pad(p['sa%d_W1' % i], ((0, Dp - (3 + C)), (0, 0)))
        f = _sa_mlp(G.reshape(B, M * NSAMPLE, Dp), f_center,
                    W1p, p['sa%d_b1' % i],
                    p['sa%d_W2' % i], p['sa%d_b2' % i],
                    p['sa%d_Wr' % i], p['sa%d_br' % i])
        xyz = new_xyz
        radius = radius * RADIUS_SCALING
    features, pred, off = _tail_heads(f.reshape(B * f.shape[1], 512), info, p)
    return (features, pred, off.reshape(-1, K_CLS, 3))
